# baseline (speedup 1.0000x reference)
import functools

import jax
import jax.numpy as jnp
from jax import lax
from jax.experimental import pallas as pl
from jax.experimental.pallas import tpu as pltpu
from jax.experimental.pallas import tpu_sc as plsc

N = 10000
E = 320000
D = 128

NC = 2
NS = 16
NW = NC * NS
K = 128
EPT = 10240
G = EPT // (4 * K)
KA = 88
NG = 30
EPTA = NG * 4 * KA
NPAD = 10112
RPT = NPAD // NS


def _make_agg():
    mesh = plsc.VectorSubcoreMesh(core_axis_name="c", subcore_axis_name="s",
                                  num_cores=NC, num_subcores=NS)
    scratch = [
        pltpu.VMEM_SHARED((NPAD, D), jnp.float32),
        pltpu.VMEM((2, 8, KA), jnp.int32),
        pltpu.VMEM((KA, D), jnp.float32),
        pltpu.VMEM((KA, D), jnp.float32),
        pltpu.VMEM((KA, D), jnp.float32),
        pltpu.VMEM((KA, D), jnp.float32),
        pltpu.SemaphoreType.DMA,
        pltpu.SemaphoreType.DMA,
        pltpu.SemaphoreType.DMA,
        pltpu.SemaphoreType.DMA,
        pltpu.SemaphoreType.DMA,
        pltpu.SemaphoreType.DMA,
        pltpu.SemaphoreType.DMA,
        pltpu.SemaphoreType.DMA,
    ]

    def body(h_hbm, idx_hbm, zeros_hbm, out_hbm,
             acc, idxv, rb0, rb1, rb2, rb3, g0, g1, g2, g3, s0, s1, s2, s3):
        cid = lax.axis_index("c")
        sid = lax.axis_index("s")
        wid = sid * NC + cid
        row0 = sid * RPT

        pltpu.sync_copy(zeros_hbm.at[pl.ds(row0, RPT)],
                        acc.at[pl.ds(row0, RPT)])
        plsc.subcore_barrier()

        rbufs = (rb0, rb1, rb2, rb3)
        gsems = (g0, g1, g2, g3)
        ssems = (s0, s1, s2, s3)

        @pl.loop(0, NG // 2)
        def _(i):
            pltpu.sync_copy(idx_hbm.at[wid, pl.ds(2 * i, 2)], idxv)
            for gg in range(2):
                gds = [pltpu.async_copy(h_hbm.at[idxv.at[gg, j]],
                                        rbufs[j], gsems[j])
                       for j in range(4)]
                sds = []
                for j in range(4):
                    gds[j].wait()
                    sds.append(pltpu.async_copy(
                        rbufs[j], acc.at[idxv.at[gg, 4 + j]], ssems[j],
                        add=True))
                for sd in sds:
                    sd.wait()

        plsc.subcore_barrier()
        pltpu.sync_copy(acc.at[pl.ds(row0, RPT)],
                        out_hbm.at[cid, pl.ds(row0, RPT)])

    return pl.kernel(body,
                     out_type=jax.ShapeDtypeStruct((NC, NPAD, D), jnp.float32),
                     mesh=mesh, scratch_types=scratch)


def _make_deg():
    import dataclasses
    mesh = plsc.VectorSubcoreMesh(core_axis_name="c", subcore_axis_name="s",
                                  num_cores=NC, num_subcores=NS)
    cp = pltpu.CompilerParams()
    if "needs_layout_passes" in pltpu.CompilerParams.__dataclass_fields__:
        cp = dataclasses.replace(cp, needs_layout_passes=False)
    scratch = [
        pltpu.VMEM((NPAD,), jnp.float32),
        pltpu.VMEM((8, K), jnp.int32),
        pltpu.SemaphoreType.DMA,
    ]

    def body(idx_hbm, deg_hbm, hist, iba, isem):
        cid = lax.axis_index("c")
        sid = lax.axis_index("s")
        wid = sid * NC + cid

        @pl.loop(0, NPAD // 16)
        def _(i):
            hist[pl.ds(i * 16, 16)] = jnp.zeros((16,), jnp.float32)

        ones16 = jnp.ones((16,), jnp.float32)

        @pl.loop(0, G)
        def _(g):
            pltpu.async_copy(idx_hbm.at[wid, g], iba, isem).wait()
            for j in range(4):
                for l in range(K // 16):
                    ids = iba[4 + j, pl.ds(l * 16, 16)]
                    plsc.addupdate_scatter(hist, [ids], ones16)

        pltpu.sync_copy(hist, deg_hbm.at[wid])

    return pl.kernel(body,
                     out_type=jax.ShapeDtypeStruct((NW, NPAD), jnp.float32),
                     mesh=mesh, compiler_params=cp, scratch_types=scratch)


_make_agg = functools.cache(_make_agg)
_make_deg = functools.cache(_make_deg)


def _tc_layer(p, pdeg, h, W_l, W_r, b, gamma, beta, W_h=None, b_h=None):
    final = W_h is not None

    def body(*refs):
        if final:
            (p_ref, pd_ref, h_ref, wl_ref, wr_ref, b_ref, g_ref, be_ref,
             wh_ref, bh_ref, o_ref) = refs
        else:
            (p_ref, pd_ref, h_ref, wl_ref, wr_ref, b_ref, g_ref, be_ref,
             o_ref) = refs
        deg = lax.dot_general(pd_ref[...], jnp.ones((NW, 1), jnp.float32),
                              (((0,), (0,)), ((), ())),
                              preferred_element_type=jnp.float32)
        deg = jnp.maximum(deg[:N], 1.0)
        agg = (p_ref[0, :N, :] + p_ref[1, :N, :]) / deg
        z = (jnp.dot(agg, wl_ref[...], preferred_element_type=jnp.float32)
             + jnp.dot(h_ref[...], wr_ref[...], preferred_element_type=jnp.float32)
             + b_ref[...])
        mu = jnp.mean(z, axis=-1, keepdims=True)
        zc = z - mu
        var = jnp.mean(zc * zc, axis=-1, keepdims=True)
        z = g_ref[...] * zc / jnp.sqrt(var + 1e-5) + be_ref[...]
        z = jnp.maximum(z, 0.0)
        if final:
            z = (jnp.dot(z, wh_ref[...], preferred_element_type=jnp.float32)
                 + bh_ref[...])
        o_ref[...] = z

    args = [p, pdeg, h, W_l, W_r, b, gamma, beta]
    if final:
        args += [W_h, b_h]
    return pl.pallas_call(
        body,
        out_shape=jax.ShapeDtypeStruct((N, D), jnp.float32),
    )(*args)


def kernel(x, edge_index, W_l0, W_r0, b0, gamma0, beta0,
           W_l1, W_r1, b1, gamma1, beta1, W_h, b_h):
    src = edge_index[0].astype(jnp.int32).reshape(NW, E // NW)
    dst = edge_index[1].astype(jnp.int32).reshape(NW, E // NW)
    pad = EPT - E // NW
    src_d = jnp.pad(src, ((0, 0), (0, pad)))
    dst_d = jnp.pad(dst, ((0, 0), (0, pad)), constant_values=N)
    packed_deg = jnp.concatenate([src_d.reshape(NW, G, 4, K),
                                  dst_d.reshape(NW, G, 4, K)], axis=2)
    pad_a = EPTA - E // NW
    src_a = jnp.pad(src, ((0, 0), (0, pad_a)))
    dst_a = jnp.pad(dst, ((0, 0), (0, pad_a)), constant_values=N)
    packed_agg = jnp.concatenate([src_a.reshape(NW, NG, 4, KA),
                                  dst_a.reshape(NW, NG, 4, KA)], axis=2)
    zeros = jnp.zeros((NPAD, D), jnp.float32)

    pdeg = _make_deg()(packed_deg)
    p0 = _make_agg()(x, packed_agg, zeros)
    h1 = _tc_layer(p0, pdeg, x, W_l0, W_r0, b0, gamma0, beta0)
    p1 = _make_agg()(h1, packed_agg, zeros)
    out = _tc_layer(p1, pdeg, h1, W_l1, W_r1, b1, gamma1, beta1, W_h, b_h)
    return out


# speedup vs baseline: 1.8222x; 1.8222x over previous
import functools

import jax
import jax.numpy as jnp
from jax import lax
from jax.experimental import pallas as pl
from jax.experimental.pallas import tpu as pltpu
from jax.experimental.pallas import tpu_sc as plsc

N = 10000
E = 320000
D = 128

NC = 2
NS = 16
NW = NC * NS
K = 128
EPT = 10240
G = EPT // (4 * K)
NPAD = 10112
RPT = NPAD // NS


def _make_agg():
    mesh = plsc.VectorSubcoreMesh(core_axis_name="c", subcore_axis_name="s",
                                  num_cores=NC, num_subcores=NS)
    scratch = [
        pltpu.VMEM_SHARED((NPAD, D), jnp.float32),
        pltpu.VMEM((8, K), jnp.int32),
        pltpu.VMEM((K, D), jnp.float32),
        pltpu.VMEM((K, D), jnp.float32),
        pltpu.SemaphoreType.DMA,
        pltpu.SemaphoreType.DMA,
        pltpu.SemaphoreType.DMA,
        pltpu.SemaphoreType.DMA,
    ]

    def body(h_hbm, idx_hbm, zeros_hbm, out_hbm,
             acc, idxv, rb0, rb1, g0, g1, s0, s1):
        cid = lax.axis_index("c")
        sid = lax.axis_index("s")
        wid = sid * NC + cid
        row0 = sid * RPT

        pltpu.sync_copy(zeros_hbm.at[pl.ds(row0, RPT)],
                        acc.at[pl.ds(row0, RPT)])
        plsc.subcore_barrier()

        rbufs = (rb0, rb1)
        gsems = (g0, g1)
        ssems = (s0, s1)

        @pl.loop(0, G)
        def _(g):
            pltpu.sync_copy(idx_hbm.at[wid, g], idxv)
            sds = [None, None]
            for j in range(4):
                b = j % 2
                if sds[b] is not None:
                    sds[b].wait()
                gd = pltpu.async_copy(h_hbm.at[idxv.at[j]],
                                      rbufs[b], gsems[b])
                gd.wait()
                sds[b] = pltpu.async_copy(
                    rbufs[b], acc.at[idxv.at[4 + j]], ssems[b], add=True)
            sds[0].wait()
            sds[1].wait()

        plsc.subcore_barrier()
        pltpu.sync_copy(acc.at[pl.ds(row0, RPT)],
                        out_hbm.at[cid, pl.ds(row0, RPT)])

    return pl.kernel(body,
                     out_type=jax.ShapeDtypeStruct((NC, NPAD, D), jnp.float32),
                     mesh=mesh, scratch_types=scratch)


def _make_deg():
    import dataclasses
    mesh = plsc.VectorSubcoreMesh(core_axis_name="c", subcore_axis_name="s",
                                  num_cores=NC, num_subcores=NS)
    cp = pltpu.CompilerParams()
    if "needs_layout_passes" in pltpu.CompilerParams.__dataclass_fields__:
        cp = dataclasses.replace(cp, needs_layout_passes=False)
    scratch = [
        pltpu.VMEM((NPAD,), jnp.float32),
        pltpu.VMEM((8, K), jnp.int32),
        pltpu.SemaphoreType.DMA,
    ]

    def body(idx_hbm, deg_hbm, hist, iba, isem):
        cid = lax.axis_index("c")
        sid = lax.axis_index("s")
        wid = sid * NC + cid

        @pl.loop(0, NPAD // 16)
        def _(i):
            hist[pl.ds(i * 16, 16)] = jnp.zeros((16,), jnp.float32)

        ones16 = jnp.ones((16,), jnp.float32)

        @pl.loop(0, G)
        def _(g):
            pltpu.async_copy(idx_hbm.at[wid, g], iba, isem).wait()
            for j in range(4):
                for l in range(K // 16):
                    ids = iba[4 + j, pl.ds(l * 16, 16)]
                    plsc.addupdate_scatter(hist, [ids], ones16)

        pltpu.sync_copy(hist, deg_hbm.at[wid])

    return pl.kernel(body,
                     out_type=jax.ShapeDtypeStruct((NW, NPAD), jnp.float32),
                     mesh=mesh, compiler_params=cp, scratch_types=scratch)


_make_agg = functools.cache(_make_agg)
_make_deg = functools.cache(_make_deg)


def _tc_layer(p, pdeg, h, W_l, W_r, b, gamma, beta, W_h=None, b_h=None):
    final = W_h is not None

    def body(*refs):
        if final:
            (p_ref, pd_ref, h_ref, wl_ref, wr_ref, b_ref, g_ref, be_ref,
             wh_ref, bh_ref, o_ref) = refs
        else:
            (p_ref, pd_ref, h_ref, wl_ref, wr_ref, b_ref, g_ref, be_ref,
             o_ref) = refs
        deg = lax.dot_general(pd_ref[...], jnp.ones((NW, 1), jnp.float32),
                              (((0,), (0,)), ((), ())),
                              preferred_element_type=jnp.float32)
        deg = jnp.maximum(deg[:N], 1.0)
        agg = (p_ref[0, :N, :] + p_ref[1, :N, :]) / deg
        z = (jnp.dot(agg, wl_ref[...], preferred_element_type=jnp.float32)
             + jnp.dot(h_ref[...], wr_ref[...], preferred_element_type=jnp.float32)
             + b_ref[...])
        mu = jnp.mean(z, axis=-1, keepdims=True)
        zc = z - mu
        var = jnp.mean(zc * zc, axis=-1, keepdims=True)
        z = g_ref[...] * zc / jnp.sqrt(var + 1e-5) + be_ref[...]
        z = jnp.maximum(z, 0.0)
        if final:
            z = (jnp.dot(z, wh_ref[...], preferred_element_type=jnp.float32)
                 + bh_ref[...])
        o_ref[...] = z

    args = [p, pdeg, h, W_l, W_r, b, gamma, beta]
    if final:
        args += [W_h, b_h]
    return pl.pallas_call(
        body,
        out_shape=jax.ShapeDtypeStruct((N, D), jnp.float32),
    )(*args)


def kernel(x, edge_index, W_l0, W_r0, b0, gamma0, beta0,
           W_l1, W_r1, b1, gamma1, beta1, W_h, b_h):
    src = edge_index[0].astype(jnp.int32).reshape(NW, E // NW)
    dst = edge_index[1].astype(jnp.int32).reshape(NW, E // NW)
    pad = EPT - E // NW
    src_d = jnp.pad(src, ((0, 0), (0, pad)))
    dst_d = jnp.pad(dst, ((0, 0), (0, pad)), constant_values=N)
    packed = jnp.concatenate([src_d.reshape(NW, G, 4, K),
                              dst_d.reshape(NW, G, 4, K)], axis=2)
    zeros = jnp.zeros((NPAD, D), jnp.float32)

    pdeg = _make_deg()(packed)
    p0 = _make_agg()(x, packed, zeros)
    h1 = _tc_layer(p0, pdeg, x, W_l0, W_r0, b0, gamma0, beta0)
    p1 = _make_agg()(h1, packed, zeros)
    out = _tc_layer(p1, pdeg, h1, W_l1, W_r1, b1, gamma1, beta1, W_h, b_h)
    return out


# speedup vs baseline: 1.8613x; 1.0215x over previous
import functools

import jax
import jax.numpy as jnp
from jax import lax
from jax.experimental import pallas as pl
from jax.experimental.pallas import tpu as pltpu
from jax.experimental.pallas import tpu_sc as plsc

N = 10000
E = 320000
D = 128

NC = 2
NS = 16
NW = NC * NS
K = 128
EPT = 10240
G = EPT // (4 * K)
NPAD = 10112
RPT = NPAD // NS


def _make_agg():
    mesh = plsc.VectorSubcoreMesh(core_axis_name="c", subcore_axis_name="s",
                                  num_cores=NC, num_subcores=NS)
    scratch = [
        pltpu.VMEM_SHARED((NPAD, D), jnp.float32),
        pltpu.VMEM((8, K), jnp.int32),
        pltpu.VMEM((K, D), jnp.float32),
        pltpu.VMEM((K, D), jnp.float32),
        pltpu.SemaphoreType.DMA,
        pltpu.SemaphoreType.DMA,
        pltpu.SemaphoreType.DMA,
        pltpu.SemaphoreType.DMA,
    ]

    def body(h_hbm, idx_hbm, zeros_hbm, out_hbm,
             acc, idxv, rb0, rb1, g0, g1, s0, s1):
        cid = lax.axis_index("c")
        sid = lax.axis_index("s")
        wid = sid * NC + cid
        row0 = sid * RPT

        pltpu.sync_copy(zeros_hbm.at[pl.ds(row0, RPT)],
                        acc.at[pl.ds(row0, RPT)])
        plsc.subcore_barrier()

        rbufs = (rb0, rb1)
        gsems = (g0, g1)
        ssems = (s0, s1)

        @pl.loop(0, G)
        def _(g):
            pltpu.sync_copy(idx_hbm.at[wid, g], idxv)
            gds = [pltpu.async_copy(h_hbm.at[idxv.at[b]], rbufs[b], gsems[b])
                   for b in range(2)]
            sds = [None, None]
            for j in range(4):
                b = j % 2
                gds[b].wait()
                sds[b] = pltpu.async_copy(
                    rbufs[b], acc.at[idxv.at[4 + j]], ssems[b], add=True)
                if j + 2 < 4:
                    sds[b].wait()
                    gds[b] = pltpu.async_copy(h_hbm.at[idxv.at[j + 2]],
                                              rbufs[b], gsems[b])
            sds[0].wait()
            sds[1].wait()

        plsc.subcore_barrier()
        pltpu.sync_copy(acc.at[pl.ds(row0, RPT)],
                        out_hbm.at[cid, pl.ds(row0, RPT)])

    return pl.kernel(body,
                     out_type=jax.ShapeDtypeStruct((NC, NPAD, D), jnp.float32),
                     mesh=mesh, scratch_types=scratch)


def _make_deg():
    import dataclasses
    mesh = plsc.VectorSubcoreMesh(core_axis_name="c", subcore_axis_name="s",
                                  num_cores=NC, num_subcores=NS)
    cp = pltpu.CompilerParams()
    if "needs_layout_passes" in pltpu.CompilerParams.__dataclass_fields__:
        cp = dataclasses.replace(cp, needs_layout_passes=False)
    scratch = [
        pltpu.VMEM((NPAD,), jnp.float32),
        pltpu.VMEM((8, K), jnp.int32),
        pltpu.SemaphoreType.DMA,
    ]

    def body(idx_hbm, deg_hbm, hist, iba, isem):
        cid = lax.axis_index("c")
        sid = lax.axis_index("s")
        wid = sid * NC + cid

        @pl.loop(0, NPAD // 16)
        def _(i):
            hist[pl.ds(i * 16, 16)] = jnp.zeros((16,), jnp.float32)

        ones16 = jnp.ones((16,), jnp.float32)

        @pl.loop(0, G)
        def _(g):
            pltpu.async_copy(idx_hbm.at[wid, g], iba, isem).wait()
            for j in range(4):
                for l in range(K // 16):
                    ids = iba[4 + j, pl.ds(l * 16, 16)]
                    plsc.addupdate_scatter(hist, [ids], ones16)

        pltpu.sync_copy(hist, deg_hbm.at[wid])

    return pl.kernel(body,
                     out_type=jax.ShapeDtypeStruct((NW, NPAD), jnp.float32),
                     mesh=mesh, compiler_params=cp, scratch_types=scratch)


_make_agg = functools.cache(_make_agg)
_make_deg = functools.cache(_make_deg)


def _tc_layer(p, pdeg, h, W_l, W_r, b, gamma, beta, W_h=None, b_h=None):
    final = W_h is not None

    def body(*refs):
        if final:
            (p_ref, pd_ref, h_ref, wl_ref, wr_ref, b_ref, g_ref, be_ref,
             wh_ref, bh_ref, o_ref) = refs
        else:
            (p_ref, pd_ref, h_ref, wl_ref, wr_ref, b_ref, g_ref, be_ref,
             o_ref) = refs
        deg = lax.dot_general(pd_ref[...], jnp.ones((NW, 1), jnp.float32),
                              (((0,), (0,)), ((), ())),
                              preferred_element_type=jnp.float32)
        deg = jnp.maximum(deg[:N], 1.0)
        agg = (p_ref[0, :N, :] + p_ref[1, :N, :]) / deg
        z = (jnp.dot(agg, wl_ref[...], preferred_element_type=jnp.float32)
             + jnp.dot(h_ref[...], wr_ref[...], preferred_element_type=jnp.float32)
             + b_ref[...])
        mu = jnp.mean(z, axis=-1, keepdims=True)
        zc = z - mu
        var = jnp.mean(zc * zc, axis=-1, keepdims=True)
        z = g_ref[...] * zc / jnp.sqrt(var + 1e-5) + be_ref[...]
        z = jnp.maximum(z, 0.0)
        if final:
            z = (jnp.dot(z, wh_ref[...], preferred_element_type=jnp.float32)
                 + bh_ref[...])
        o_ref[...] = z

    args = [p, pdeg, h, W_l, W_r, b, gamma, beta]
    if final:
        args += [W_h, b_h]
    return pl.pallas_call(
        body,
        out_shape=jax.ShapeDtypeStruct((N, D), jnp.float32),
    )(*args)


def kernel(x, edge_index, W_l0, W_r0, b0, gamma0, beta0,
           W_l1, W_r1, b1, gamma1, beta1, W_h, b_h):
    src = edge_index[0].astype(jnp.int32).reshape(NW, E // NW)
    dst = edge_index[1].astype(jnp.int32).reshape(NW, E // NW)
    pad = EPT - E // NW
    src_d = jnp.pad(src, ((0, 0), (0, pad)))
    dst_d = jnp.pad(dst, ((0, 0), (0, pad)), constant_values=N)
    packed = jnp.concatenate([src_d.reshape(NW, G, 4, K),
                              dst_d.reshape(NW, G, 4, K)], axis=2)
    zeros = jnp.zeros((NPAD, D), jnp.float32)

    pdeg = _make_deg()(packed)
    p0 = _make_agg()(x, packed, zeros)
    h1 = _tc_layer(p0, pdeg, x, W_l0, W_r0, b0, gamma0, beta0)
    p1 = _make_agg()(h1, packed, zeros)
    out = _tc_layer(p1, pdeg, h1, W_l1, W_r1, b1, gamma1, beta1, W_h, b_h)
    return out


# speedup vs baseline: 1.8964x; 1.0189x over previous
import functools

import jax
import jax.numpy as jnp
from jax import lax
from jax.experimental import pallas as pl
from jax.experimental.pallas import tpu as pltpu
from jax.experimental.pallas import tpu_sc as plsc

N = 10000
E = 320000
D = 128

NC = 2
NS = 16
NW = NC * NS
K = 128
EPT = 10240
G = EPT // (4 * K)
NPAD = 10112
RPT = NPAD // NS


def _make_agg():
    mesh = plsc.VectorSubcoreMesh(core_axis_name="c", subcore_axis_name="s",
                                  num_cores=NC, num_subcores=NS)
    scratch = [
        pltpu.VMEM_SHARED((NPAD, D), jnp.float32),
        pltpu.VMEM((G // 2, 8, K), jnp.int32),
        pltpu.VMEM((K, D), jnp.float32),
        pltpu.VMEM((K, D), jnp.float32),
        pltpu.SemaphoreType.DMA,
        pltpu.SemaphoreType.DMA,
        pltpu.SemaphoreType.DMA,
        pltpu.SemaphoreType.DMA,
    ]

    def body(h_hbm, idx_hbm, zeros_hbm, out_hbm,
             acc, idxv, rb0, rb1, g0, g1, s0, s1):
        cid = lax.axis_index("c")
        sid = lax.axis_index("s")
        wid = sid * NC + cid
        row0 = sid * RPT

        pltpu.sync_copy(zeros_hbm.at[pl.ds(row0, RPT)],
                        acc.at[pl.ds(row0, RPT)])
        plsc.subcore_barrier()

        rbufs = (rb0, rb1)
        gsems = (g0, g1)
        ssems = (s0, s1)

        for h in range(2):
            pltpu.sync_copy(idx_hbm.at[wid, pl.ds(h * (G // 2), G // 2)],
                            idxv)

            @pl.loop(0, G // 2)
            def _(g):
                gds = [pltpu.async_copy(h_hbm.at[idxv.at[g, b]],
                                        rbufs[b], gsems[b])
                       for b in range(2)]
                sds = [None, None]
                for j in range(4):
                    b = j % 2
                    gds[b].wait()
                    sds[b] = pltpu.async_copy(
                        rbufs[b], acc.at[idxv.at[g, 4 + j]], ssems[b],
                        add=True)
                    if j + 2 < 4:
                        sds[b].wait()
                        gds[b] = pltpu.async_copy(
                            h_hbm.at[idxv.at[g, j + 2]], rbufs[b], gsems[b])
                sds[0].wait()
                sds[1].wait()

        plsc.subcore_barrier()
        pltpu.sync_copy(acc.at[pl.ds(row0, RPT)],
                        out_hbm.at[cid, pl.ds(row0, RPT)])

    return pl.kernel(body,
                     out_type=jax.ShapeDtypeStruct((NC, NPAD, D), jnp.float32),
                     mesh=mesh, scratch_types=scratch)


def _make_deg():
    import dataclasses
    mesh = plsc.VectorSubcoreMesh(core_axis_name="c", subcore_axis_name="s",
                                  num_cores=NC, num_subcores=NS)
    cp = pltpu.CompilerParams()
    if "needs_layout_passes" in pltpu.CompilerParams.__dataclass_fields__:
        cp = dataclasses.replace(cp, needs_layout_passes=False)
    scratch = [
        pltpu.VMEM((NPAD,), jnp.float32),
        pltpu.VMEM((8, K), jnp.int32),
        pltpu.SemaphoreType.DMA,
    ]

    def body(idx_hbm, deg_hbm, hist, iba, isem):
        cid = lax.axis_index("c")
        sid = lax.axis_index("s")
        wid = sid * NC + cid

        @pl.loop(0, NPAD // 16)
        def _(i):
            hist[pl.ds(i * 16, 16)] = jnp.zeros((16,), jnp.float32)

        ones16 = jnp.ones((16,), jnp.float32)

        @pl.loop(0, G)
        def _(g):
            pltpu.async_copy(idx_hbm.at[wid, g], iba, isem).wait()
            for j in range(4):
                for l in range(K // 16):
                    ids = iba[4 + j, pl.ds(l * 16, 16)]
                    plsc.addupdate_scatter(hist, [ids], ones16)

        pltpu.sync_copy(hist, deg_hbm.at[wid])

    return pl.kernel(body,
                     out_type=jax.ShapeDtypeStruct((NW, NPAD), jnp.float32),
                     mesh=mesh, compiler_params=cp, scratch_types=scratch)


_make_agg = functools.cache(_make_agg)
_make_deg = functools.cache(_make_deg)


def _tc_layer(p, pdeg, h, W_l, W_r, b, gamma, beta, W_h=None, b_h=None):
    final = W_h is not None

    def body(*refs):
        if final:
            (p_ref, pd_ref, h_ref, wl_ref, wr_ref, b_ref, g_ref, be_ref,
             wh_ref, bh_ref, o_ref) = refs
        else:
            (p_ref, pd_ref, h_ref, wl_ref, wr_ref, b_ref, g_ref, be_ref,
             o_ref) = refs
        deg = lax.dot_general(pd_ref[...], jnp.ones((NW, 1), jnp.float32),
                              (((0,), (0,)), ((), ())),
                              preferred_element_type=jnp.float32)
        deg = jnp.maximum(deg[:N], 1.0)
        agg = (p_ref[0, :N, :] + p_ref[1, :N, :]) / deg
        z = (jnp.dot(agg, wl_ref[...], preferred_element_type=jnp.float32)
             + jnp.dot(h_ref[...], wr_ref[...], preferred_element_type=jnp.float32)
             + b_ref[...])
        mu = jnp.mean(z, axis=-1, keepdims=True)
        zc = z - mu
        var = jnp.mean(zc * zc, axis=-1, keepdims=True)
        z = g_ref[...] * zc / jnp.sqrt(var + 1e-5) + be_ref[...]
        z = jnp.maximum(z, 0.0)
        if final:
            z = (jnp.dot(z, wh_ref[...], preferred_element_type=jnp.float32)
                 + bh_ref[...])
        o_ref[...] = z

    args = [p, pdeg, h, W_l, W_r, b, gamma, beta]
    if final:
        args += [W_h, b_h]
    return pl.pallas_call(
        body,
        out_shape=jax.ShapeDtypeStruct((N, D), jnp.float32),
    )(*args)


def kernel(x, edge_index, W_l0, W_r0, b0, gamma0, beta0,
           W_l1, W_r1, b1, gamma1, beta1, W_h, b_h):
    src = edge_index[0].astype(jnp.int32).reshape(NW, E // NW)
    dst = edge_index[1].astype(jnp.int32).reshape(NW, E // NW)
    pad = EPT - E // NW
    src_d = jnp.pad(src, ((0, 0), (0, pad)))
    dst_d = jnp.pad(dst, ((0, 0), (0, pad)), constant_values=N)
    packed = jnp.concatenate([src_d.reshape(NW, G, 4, K),
                              dst_d.reshape(NW, G, 4, K)], axis=2)
    zeros = jnp.zeros((NPAD, D), jnp.float32)

    pdeg = _make_deg()(packed)
    p0 = _make_agg()(x, packed, zeros)
    h1 = _tc_layer(p0, pdeg, x, W_l0, W_r0, b0, gamma0, beta0)
    p1 = _make_agg()(h1, packed, zeros)
    out = _tc_layer(p1, pdeg, h1, W_l1, W_r1, b1, gamma1, beta1, W_h, b_h)
    return out
